# Initial kernel scaffold; baseline (speedup 1.0000x reference)
import jax
import jax.numpy as jnp
from jax.experimental import pallas as pl


def kernel(x, edge_index, eps0, Wa0, ba0, gm0, bm0, Wb0, bb0, g0, b0, eps1, Wa1, ba1, gm1, bm1, Wb1, bb1, g1, b1):
    raise NotImplementedError("write your pallas kernel here")


# speedup vs baseline: 5.2962x; 5.2962x over previous
import functools

import jax
import jax.numpy as jnp
from jax import lax
from jax.experimental import pallas as pl
from jax.experimental.pallas import tpu as pltpu
from jax.experimental.pallas import tpu_sc as plsc

N = 10000
D = 128
E = 320000

CHUNK = 128
NCHUNK = E // CHUNK
NC, NS = 2, 16
NW = NC * NS
STEPS = (NCHUNK + NW - 1) // NW
ROWS_PAD = 640
N_PAD = ROWS_PAD * NS

BLK = 2000
NBLK = N // BLK


def _sc_agg(x, src2d, dst2d):
    mesh = plsc.VectorSubcoreMesh(core_axis_name="c", subcore_axis_name="s")

    @functools.partial(
        pl.kernel,
        out_type=jax.ShapeDtypeStruct((NC, N, D), jnp.float32),
        mesh=mesh,
        scratch_types=[
            pltpu.VMEM((2, CHUNK), jnp.int32),
            pltpu.VMEM((2, CHUNK), jnp.int32),
            pltpu.VMEM((CHUNK, D), jnp.float32),
            pltpu.VMEM((CHUNK, D), jnp.float32),
            pltpu.VMEM_SHARED((N_PAD, D), jnp.float32),
        ],
    )
    def k(x_hbm, src_hbm, dst_hbm, out_hbm, sidx, didx, rows, zbuf, agg):
        cid = lax.axis_index("c")
        sid = lax.axis_index("s")

        @pl.loop(0, CHUNK)
        def _(r):
            @pl.loop(0, D, step=16)
            def _(c):
                zbuf[r, pl.ds(c, 16)] = jnp.zeros((16,), jnp.float32)

        zbase = sid * ROWS_PAD

        @pl.loop(0, ROWS_PAD // CHUNK)
        def _(t):
            pltpu.sync_copy(zbuf, agg.at[pl.ds(zbase + t * CHUNK, CHUNK)])

        plsc.subcore_barrier()

        wid = sid * NC + cid

        @pl.loop(0, STEPS)
        def _(t):
            chunk = t * NW + wid

            @pl.when(chunk < NCHUNK)
            def _():
                pltpu.sync_copy(src_hbm.at[chunk], sidx.at[0])
                pltpu.sync_copy(dst_hbm.at[chunk], didx.at[0])
                pltpu.sync_copy(x_hbm.at[sidx.at[0]], rows)
                pltpu.sync_copy(rows, agg.at[didx.at[0]], add=True)

        plsc.subcore_barrier()

        obase = sid * ROWS_PAD

        @pl.when(sid < NS - 1)
        def _():
            pltpu.sync_copy(agg.at[pl.ds(obase, ROWS_PAD)],
                            out_hbm.at[cid, pl.ds(obase, ROWS_PAD)])

        @pl.when(sid == NS - 1)
        def _():
            pltpu.sync_copy(agg.at[pl.ds(obase, N - (NS - 1) * ROWS_PAD)],
                            out_hbm.at[cid, pl.ds(obase, N - (NS - 1) * ROWS_PAD)])

    return k(x, src2d, dst2d)


def _dot(a, b):
    return lax.dot_general(a, b, (((1,), (0,)), ((), ())),
                           preferred_element_type=jnp.float32,
                           precision=lax.Precision.HIGHEST)


def _k1_body(eps_ref, o_ref, a0_ref, a1_ref, w_ref, b_ref, h_ref, st_ref):
    i = pl.program_id(0)
    pre = (1.0 + eps_ref[0, 0]) * o_ref[...] + a0_ref[...] + a1_ref[...]
    h = _dot(pre, w_ref[...]) + b_ref[...]
    h_ref[...] = h

    @pl.when(i == 0)
    def _():
        st_ref[...] = jnp.zeros_like(st_ref)

    st_ref[0:1, :] += jnp.sum(h, axis=0, keepdims=True)
    st_ref[1:2, :] += jnp.sum(h * h, axis=0, keepdims=True)


def _k2_body(h_ref, st_ref, gm_ref, bm_ref, w_ref, b_ref, h2_ref, st2_ref):
    i = pl.program_id(0)
    mu = st_ref[0:1, :] * (1.0 / N)
    var = st_ref[1:2, :] * (1.0 / N) - mu * mu
    inv = lax.rsqrt(var + 1e-5)
    hn = (h_ref[...] - mu) * (inv * gm_ref[...]) + bm_ref[...]
    hn = jnp.maximum(hn, 0.0)
    h2 = _dot(hn, w_ref[...]) + b_ref[...]
    h2_ref[...] = h2

    @pl.when(i == 0)
    def _():
        st2_ref[...] = jnp.zeros_like(st2_ref)

    st2_ref[0:1, :] += jnp.sum(h2, axis=0, keepdims=True)
    st2_ref[1:2, :] += jnp.sum(h2 * h2, axis=0, keepdims=True)


def _k3_body(h_ref, st_ref, g_ref, b_ref, o_ref):
    mu = st_ref[0:1, :] * (1.0 / N)
    var = st_ref[1:2, :] * (1.0 / N) - mu * mu
    inv = lax.rsqrt(var + 1e-5)
    hn = (h_ref[...] - mu) * (inv * g_ref[...]) + b_ref[...]
    o_ref[...] = jnp.maximum(hn, 0.0)


_row_spec = pl.BlockSpec((BLK, D), lambda i: (i, 0))


def _vspec(shape):
    return pl.BlockSpec(shape, lambda i: tuple(0 for _ in shape))


def _gin_layer(o, a0, a1, eps, Wa, ba, gm, bm, Wb, bb, g, b):
    eps2 = jnp.reshape(eps, (1, 1))
    ba2, gm2, bm2 = ba.reshape(1, D), gm.reshape(1, D), bm.reshape(1, D)
    bb2, g2, b2 = bb.reshape(1, D), g.reshape(1, D), b.reshape(1, D)

    h1, st1 = pl.pallas_call(
        _k1_body,
        grid=(NBLK,),
        in_specs=[
            pl.BlockSpec(memory_space=pltpu.SMEM),
            _row_spec, _row_spec, _row_spec,
            _vspec((D, D)), _vspec((1, D)),
        ],
        out_specs=[_row_spec, _vspec((8, D))],
        out_shape=[jax.ShapeDtypeStruct((N, D), jnp.float32),
                   jax.ShapeDtypeStruct((8, D), jnp.float32)],
    )(eps2, o, a0, a1, Wa, ba2)

    h2, st2 = pl.pallas_call(
        _k2_body,
        grid=(NBLK,),
        in_specs=[_row_spec, _vspec((8, D)), _vspec((1, D)), _vspec((1, D)),
                  _vspec((D, D)), _vspec((1, D))],
        out_specs=[_row_spec, _vspec((8, D))],
        out_shape=[jax.ShapeDtypeStruct((N, D), jnp.float32),
                   jax.ShapeDtypeStruct((8, D), jnp.float32)],
    )(h1, st1, gm2, bm2, Wb, bb2)

    o_next = pl.pallas_call(
        _k3_body,
        grid=(NBLK,),
        in_specs=[_row_spec, _vspec((8, D)), _vspec((1, D)), _vspec((1, D))],
        out_specs=_row_spec,
        out_shape=jax.ShapeDtypeStruct((N, D), jnp.float32),
    )(h2, st2, g2, b2)
    return o_next


def kernel(x, edge_index, eps0, Wa0, ba0, gm0, bm0, Wb0, bb0, g0, b0,
           eps1, Wa1, ba1, gm1, bm1, Wb1, bb1, g1, b1):
    src2d = edge_index[0].reshape(NCHUNK, CHUNK)
    dst2d = edge_index[1].reshape(NCHUNK, CHUNK)

    aggp0 = _sc_agg(x, src2d, dst2d)
    o1 = _gin_layer(x, aggp0[0], aggp0[1], eps0, Wa0, ba0, gm0, bm0,
                    Wb0, bb0, g0, b0)
    aggp1 = _sc_agg(o1, src2d, dst2d)
    o2 = _gin_layer(o1, aggp1[0], aggp1[1], eps1, Wa1, ba1, gm1, bm1,
                    Wb1, bb1, g1, b1)
    return jnp.concatenate([x, o1, o2], axis=1)


# speedup vs baseline: 8.5892x; 1.6218x over previous
import functools

import jax
import jax.numpy as jnp
from jax import lax
from jax.experimental import pallas as pl
from jax.experimental.pallas import tpu as pltpu
from jax.experimental.pallas import tpu_sc as plsc

N = 10000
D = 128
E = 320000

CHUNK = 128
NC, NS = 2, 16
NW = NC * NS
STEPS = 80
PSTEPS = 40
E_PAD = NW * STEPS * CHUNK
ROWS_PAD = 640
N_PAD = ROWS_PAD * NS

BLK = 2000
NBLK = N // BLK


def _sc_agg(x, src3d, dst3d):
    mesh = plsc.VectorSubcoreMesh(core_axis_name="c", subcore_axis_name="s")

    @functools.partial(
        pl.kernel,
        out_type=jax.ShapeDtypeStruct((NC, N, D), jnp.float32),
        mesh=mesh,
        scratch_types=[
            pltpu.VMEM((PSTEPS, CHUNK), jnp.int32),
            pltpu.VMEM((PSTEPS, CHUNK), jnp.int32),
            pltpu.VMEM((CHUNK, D), jnp.float32),
            pltpu.VMEM((CHUNK, D), jnp.float32),
            pltpu.VMEM_SHARED((N_PAD, D), jnp.float32),
            pltpu.SemaphoreType.DMA,
            pltpu.SemaphoreType.DMA,
            pltpu.SemaphoreType.DMA,
        ],
    )
    def k(x_hbm, src_hbm, dst_hbm, out_hbm, sidx, didx, bufa, bufb, agg,
          isem, sema, semb):
        cid = lax.axis_index("c")
        sid = lax.axis_index("s")
        wid = sid * NC + cid

        for p in range(STEPS // PSTEPS):
            pltpu.async_copy(src_hbm.at[wid, pl.ds(p * PSTEPS, PSTEPS)],
                             sidx, isem)
            pltpu.async_copy(dst_hbm.at[wid, pl.ds(p * PSTEPS, PSTEPS)],
                             didx, isem)

            if p == 0:
                @pl.loop(0, CHUNK)
                def _(r):
                    @pl.loop(0, D, step=16)
                    def _(c):
                        bufa[r, pl.ds(c, 16)] = jnp.zeros((16,), jnp.float32)

                zbase = sid * ROWS_PAD

                @pl.loop(0, ROWS_PAD // CHUNK)
                def _(t):
                    pltpu.sync_copy(bufa, agg.at[pl.ds(zbase + t * CHUNK,
                                                       CHUNK)])

            pltpu.make_async_copy(src_hbm.at[wid, pl.ds(p * PSTEPS, PSTEPS)],
                                  sidx, isem).wait()
            pltpu.make_async_copy(dst_hbm.at[wid, pl.ds(p * PSTEPS, PSTEPS)],
                                  didx, isem).wait()
            if p == 0:
                plsc.subcore_barrier()

            pltpu.async_copy(x_hbm.at[sidx.at[0]], bufa, sema)

            @pl.loop(0, PSTEPS, step=2)
            def _(t):
                pltpu.make_async_copy(x_hbm.at[sidx.at[t]], bufa, sema).wait()
                pltpu.async_copy(x_hbm.at[sidx.at[t + 1]], bufb, semb)
                pltpu.sync_copy(bufa, agg.at[didx.at[t]], add=True)
                pltpu.make_async_copy(x_hbm.at[sidx.at[t + 1]], bufb,
                                      semb).wait()

                @pl.when(t + 2 < PSTEPS)
                def _():
                    pltpu.async_copy(x_hbm.at[sidx.at[t + 2]], bufa, sema)

                pltpu.sync_copy(bufb, agg.at[didx.at[t + 1]], add=True)

        plsc.subcore_barrier()

        obase = sid * ROWS_PAD

        @pl.when(sid < NS - 1)
        def _():
            pltpu.sync_copy(agg.at[pl.ds(obase, ROWS_PAD)],
                            out_hbm.at[cid, pl.ds(obase, ROWS_PAD)])

        @pl.when(sid == NS - 1)
        def _():
            pltpu.sync_copy(agg.at[pl.ds(obase, N - (NS - 1) * ROWS_PAD)],
                            out_hbm.at[cid, pl.ds(obase, N - (NS - 1) * ROWS_PAD)])

    return k(x, src3d, dst3d)


def _dot(a, b):
    return lax.dot_general(a, b, (((1,), (0,)), ((), ())),
                           preferred_element_type=jnp.float32,
                           precision=lax.Precision.HIGHEST)


def _k1_body(eps_ref, o_ref, a0_ref, a1_ref, w_ref, b_ref, h_ref, st_ref):
    i = pl.program_id(0)
    pre = (1.0 + eps_ref[0, 0]) * o_ref[...] + a0_ref[...] + a1_ref[...]
    h = _dot(pre, w_ref[...]) + b_ref[...]
    h_ref[...] = h

    @pl.when(i == 0)
    def _():
        st_ref[...] = jnp.zeros_like(st_ref)

    st_ref[0:1, :] += jnp.sum(h, axis=0, keepdims=True)
    st_ref[1:2, :] += jnp.sum(h * h, axis=0, keepdims=True)


def _k2_body(h_ref, st_ref, gm_ref, bm_ref, w_ref, b_ref, h2_ref, st2_ref):
    i = pl.program_id(0)
    mu = st_ref[0:1, :] * (1.0 / N)
    var = st_ref[1:2, :] * (1.0 / N) - mu * mu
    inv = lax.rsqrt(var + 1e-5)
    hn = (h_ref[...] - mu) * (inv * gm_ref[...]) + bm_ref[...]
    hn = jnp.maximum(hn, 0.0)
    h2 = _dot(hn, w_ref[...]) + b_ref[...]
    h2_ref[...] = h2

    @pl.when(i == 0)
    def _():
        st2_ref[...] = jnp.zeros_like(st2_ref)

    st2_ref[0:1, :] += jnp.sum(h2, axis=0, keepdims=True)
    st2_ref[1:2, :] += jnp.sum(h2 * h2, axis=0, keepdims=True)


def _k3_body(h_ref, st_ref, g_ref, b_ref, o_ref):
    mu = st_ref[0:1, :] * (1.0 / N)
    var = st_ref[1:2, :] * (1.0 / N) - mu * mu
    inv = lax.rsqrt(var + 1e-5)
    hn = (h_ref[...] - mu) * (inv * g_ref[...]) + b_ref[...]
    o_ref[...] = jnp.maximum(hn, 0.0)


_row_spec = pl.BlockSpec((BLK, D), lambda i: (i, 0))


def _vspec(shape):
    return pl.BlockSpec(shape, lambda i: tuple(0 for _ in shape))


def _gin_layer(o, a0, a1, eps, Wa, ba, gm, bm, Wb, bb, g, b):
    eps2 = jnp.reshape(eps, (1, 1))
    ba2, gm2, bm2 = ba.reshape(1, D), gm.reshape(1, D), bm.reshape(1, D)
    bb2, g2, b2 = bb.reshape(1, D), g.reshape(1, D), b.reshape(1, D)

    h1, st1 = pl.pallas_call(
        _k1_body,
        grid=(NBLK,),
        in_specs=[
            pl.BlockSpec(memory_space=pltpu.SMEM),
            _row_spec, _row_spec, _row_spec,
            _vspec((D, D)), _vspec((1, D)),
        ],
        out_specs=[_row_spec, _vspec((8, D))],
        out_shape=[jax.ShapeDtypeStruct((N, D), jnp.float32),
                   jax.ShapeDtypeStruct((8, D), jnp.float32)],
    )(eps2, o, a0, a1, Wa, ba2)

    h2, st2 = pl.pallas_call(
        _k2_body,
        grid=(NBLK,),
        in_specs=[_row_spec, _vspec((8, D)), _vspec((1, D)), _vspec((1, D)),
                  _vspec((D, D)), _vspec((1, D))],
        out_specs=[_row_spec, _vspec((8, D))],
        out_shape=[jax.ShapeDtypeStruct((N, D), jnp.float32),
                   jax.ShapeDtypeStruct((8, D), jnp.float32)],
    )(h1, st1, gm2, bm2, Wb, bb2)

    o_next = pl.pallas_call(
        _k3_body,
        grid=(NBLK,),
        in_specs=[_row_spec, _vspec((8, D)), _vspec((1, D)), _vspec((1, D))],
        out_specs=_row_spec,
        out_shape=jax.ShapeDtypeStruct((N, D), jnp.float32),
    )(h2, st2, g2, b2)
    return o_next


def kernel(x, edge_index, eps0, Wa0, ba0, gm0, bm0, Wb0, bb0, g0, b0,
           eps1, Wa1, ba1, gm1, bm1, Wb1, bb1, g1, b1):
    pad = E_PAD - E
    iot = jnp.arange(pad, dtype=jnp.int32)
    src_p = jnp.concatenate([edge_index[0].astype(jnp.int32), iot % N])
    dst_p = jnp.concatenate([edge_index[1].astype(jnp.int32),
                             N + (iot % (N_PAD - N))])
    src3d = src_p.reshape(NW, STEPS, CHUNK)
    dst3d = dst_p.reshape(NW, STEPS, CHUNK)

    aggp0 = _sc_agg(x, src3d, dst3d)
    o1 = _gin_layer(x, aggp0[0], aggp0[1], eps0, Wa0, ba0, gm0, bm0,
                    Wb0, bb0, g0, b0)
    aggp1 = _sc_agg(o1, src3d, dst3d)
    o2 = _gin_layer(o1, aggp1[0], aggp1[1], eps1, Wa1, ba1, gm1, bm1,
                    Wb1, bb1, g1, b1)
    return jnp.concatenate([x, o1, o2], axis=1)


# speedup vs baseline: 9.1860x; 1.0695x over previous
import functools

import jax
import jax.numpy as jnp
from jax import lax
from jax.experimental import pallas as pl
from jax.experimental.pallas import tpu as pltpu
from jax.experimental.pallas import tpu_sc as plsc

N = 10000
D = 128
E = 320000

CHUNK = 128
NC, NS = 2, 16
NW = NC * NS
STEPS = 80
PSTEPS = 40
E_PAD = NW * STEPS * CHUNK
ROWS_PAD = 640
N_PAD = ROWS_PAD * NS

BLK = 2000
NBLK = N // BLK


def _sc_agg(x, src3d, dst3d):
    mesh = plsc.VectorSubcoreMesh(core_axis_name="c", subcore_axis_name="s")

    @functools.partial(
        pl.kernel,
        out_type=jax.ShapeDtypeStruct((NC, N, D), jnp.float32),
        mesh=mesh,
        scratch_types=[
            pltpu.VMEM((PSTEPS, CHUNK), jnp.int32),
            pltpu.VMEM((PSTEPS, CHUNK), jnp.int32),
            pltpu.VMEM((CHUNK, D), jnp.float32),
            pltpu.VMEM((CHUNK, D), jnp.float32),
            pltpu.VMEM_SHARED((N_PAD, D), jnp.float32),
            pltpu.SemaphoreType.DMA,
            pltpu.SemaphoreType.DMA,
            pltpu.SemaphoreType.DMA,
        ],
    )
    def k(x_hbm, src_hbm, dst_hbm, out_hbm, sidx, didx, bufa, bufb, agg,
          isem, sema, semb):
        cid = lax.axis_index("c")
        sid = lax.axis_index("s")
        wid = sid * NC + cid

        for p in range(STEPS // PSTEPS):
            pltpu.async_copy(src_hbm.at[wid, pl.ds(p * PSTEPS, PSTEPS)],
                             sidx, isem)
            pltpu.async_copy(dst_hbm.at[wid, pl.ds(p * PSTEPS, PSTEPS)],
                             didx, isem)

            if p == 0:
                @pl.loop(0, CHUNK)
                def _(r):
                    @pl.loop(0, D, step=16)
                    def _(c):
                        bufa[r, pl.ds(c, 16)] = jnp.zeros((16,), jnp.float32)

                zbase = sid * ROWS_PAD

                @pl.loop(0, ROWS_PAD // CHUNK)
                def _(t):
                    pltpu.sync_copy(bufa, agg.at[pl.ds(zbase + t * CHUNK,
                                                       CHUNK)])

            pltpu.make_async_copy(src_hbm.at[wid, pl.ds(p * PSTEPS, PSTEPS)],
                                  sidx, isem).wait()
            pltpu.make_async_copy(dst_hbm.at[wid, pl.ds(p * PSTEPS, PSTEPS)],
                                  didx, isem).wait()
            if p == 0:
                plsc.subcore_barrier()

            pltpu.async_copy(x_hbm.at[sidx.at[0]], bufa, sema)

            @pl.loop(0, PSTEPS, step=2)
            def _(t):
                pltpu.make_async_copy(x_hbm.at[sidx.at[t]], bufa, sema).wait()
                pltpu.async_copy(x_hbm.at[sidx.at[t + 1]], bufb, semb)
                pltpu.sync_copy(bufa, agg.at[didx.at[t]], add=True)
                pltpu.make_async_copy(x_hbm.at[sidx.at[t + 1]], bufb,
                                      semb).wait()

                @pl.when(t + 2 < PSTEPS)
                def _():
                    pltpu.async_copy(x_hbm.at[sidx.at[t + 2]], bufa, sema)

                pltpu.sync_copy(bufb, agg.at[didx.at[t + 1]], add=True)

        plsc.subcore_barrier()

        obase = sid * ROWS_PAD

        @pl.when(sid < NS - 1)
        def _():
            pltpu.sync_copy(agg.at[pl.ds(obase, ROWS_PAD)],
                            out_hbm.at[cid, pl.ds(obase, ROWS_PAD)])

        @pl.when(sid == NS - 1)
        def _():
            pltpu.sync_copy(agg.at[pl.ds(obase, N - (NS - 1) * ROWS_PAD)],
                            out_hbm.at[cid, pl.ds(obase, N - (NS - 1) * ROWS_PAD)])

    return k(x, src3d, dst3d)


def _dot(a, b):
    return lax.dot_general(a, b, (((1,), (0,)), ((), ())),
                           preferred_element_type=jnp.float32,
                           precision=lax.Precision.HIGHEST)


def _phases01(p, i, eps_ref, o_ref, a0_ref, a1_ref, Wa_ref, ba_ref, gm_ref,
              bm_ref, Wb_ref, bb_ref, h1_ref, h2_ref, st_ref):
    @pl.when(jnp.logical_and(p == 0, i == 0))
    def _():
        st_ref[...] = jnp.zeros_like(st_ref)

    @pl.when(p == 0)
    def _():
        pre = (1.0 + eps_ref[0, 0]) * o_ref[...] + a0_ref[...] + a1_ref[...]
        h = _dot(pre, Wa_ref[...]) + ba_ref[...]
        h1_ref[pl.ds(i * BLK, BLK), :] = h
        st_ref[0:1, :] += jnp.sum(h, axis=0, keepdims=True)
        st_ref[1:2, :] += jnp.sum(h * h, axis=0, keepdims=True)

    @pl.when(p == 1)
    def _():
        mu = st_ref[0:1, :] * (1.0 / N)
        var = st_ref[1:2, :] * (1.0 / N) - mu * mu
        inv = lax.rsqrt(var + 1e-5)
        hn = (h1_ref[pl.ds(i * BLK, BLK), :] - mu) * (inv * gm_ref[...])
        hn = jnp.maximum(hn + bm_ref[...], 0.0)
        h2 = _dot(hn, Wb_ref[...]) + bb_ref[...]
        h2_ref[pl.ds(i * BLK, BLK), :] = h2
        st_ref[2:3, :] += jnp.sum(h2, axis=0, keepdims=True)
        st_ref[3:4, :] += jnp.sum(h2 * h2, axis=0, keepdims=True)


def _bn2(h2, st_ref, g_ref, b_ref):
    mu = st_ref[2:3, :] * (1.0 / N)
    var = st_ref[3:4, :] * (1.0 / N) - mu * mu
    inv = lax.rsqrt(var + 1e-5)
    return jnp.maximum((h2 - mu) * (inv * g_ref[...]) + b_ref[...], 0.0)


def _layer0_body(eps_ref, o_ref, a0_ref, a1_ref, Wa_ref, ba_ref, gm_ref,
                 bm_ref, Wb_ref, bb_ref, g_ref, b_ref, out_ref,
                 h1_ref, h2_ref, st_ref):
    p, i = pl.program_id(0), pl.program_id(1)
    _phases01(p, i, eps_ref, o_ref, a0_ref, a1_ref, Wa_ref, ba_ref, gm_ref,
              bm_ref, Wb_ref, bb_ref, h1_ref, h2_ref, st_ref)

    @pl.when(p == 2)
    def _():
        out_ref[...] = _bn2(h2_ref[pl.ds(i * BLK, BLK), :], st_ref,
                            g_ref, b_ref)


def _layer1_body(eps_ref, x_ref, o_ref, a0_ref, a1_ref, Wa_ref, ba_ref,
                 gm_ref, bm_ref, Wb_ref, bb_ref, g_ref, b_ref, out_ref,
                 h1_ref, h2_ref, st_ref):
    p, i = pl.program_id(0), pl.program_id(1)
    _phases01(p, i, eps_ref, o_ref, a0_ref, a1_ref, Wa_ref, ba_ref, gm_ref,
              bm_ref, Wb_ref, bb_ref, h1_ref, h2_ref, st_ref)

    @pl.when(p == 2)
    def _():
        out_ref[:, 0:D] = x_ref[...]
        out_ref[:, D:2 * D] = o_ref[...]
        out_ref[:, 2 * D:3 * D] = _bn2(h2_ref[pl.ds(i * BLK, BLK), :],
                                       st_ref, g_ref, b_ref)


def _vspec(shape):
    return pl.BlockSpec(shape, lambda p, i: tuple(0 for _ in shape))


_scratch = [pltpu.VMEM((N, D), jnp.float32),
            pltpu.VMEM((N, D), jnp.float32),
            pltpu.VMEM((8, D), jnp.float32)]


def _prep(eps, vecs):
    return (jnp.reshape(eps, (1, 1)),) + tuple(v.reshape(1, D) for v in vecs)


def _gin_layer0(o, a0, a1, eps, Wa, ba, gm, bm, Wb, bb, g, b):
    eps2, ba2, gm2, bm2, bb2, g2, b2 = _prep(eps, (ba, gm, bm, bb, g, b))
    p0_spec = pl.BlockSpec((BLK, D), lambda p, i: (jnp.where(p == 0, i, 0), 0))
    return pl.pallas_call(
        _layer0_body,
        grid=(3, NBLK),
        in_specs=[pl.BlockSpec(memory_space=pltpu.SMEM),
                  p0_spec, p0_spec, p0_spec,
                  _vspec((D, D)), _vspec((1, D)), _vspec((1, D)),
                  _vspec((1, D)), _vspec((D, D)), _vspec((1, D)),
                  _vspec((1, D)), _vspec((1, D))],
        out_specs=pl.BlockSpec((BLK, D),
                               lambda p, i: (jnp.where(p == 2, i, 0), 0)),
        out_shape=jax.ShapeDtypeStruct((N, D), jnp.float32),
        scratch_shapes=_scratch,
    )(eps2, o, a0, a1, Wa, ba2, gm2, bm2, Wb, bb2, g2, b2)


def _gin_layer1(x, o, a0, a1, eps, Wa, ba, gm, bm, Wb, bb, g, b):
    eps2, ba2, gm2, bm2, bb2, g2, b2 = _prep(eps, (ba, gm, bm, bb, g, b))
    p0_spec = pl.BlockSpec((BLK, D), lambda p, i: (jnp.where(p == 0, i, 0), 0))
    p2_spec = pl.BlockSpec((BLK, D), lambda p, i: (jnp.where(p == 2, i, 0), 0))
    p02_spec = pl.BlockSpec((BLK, D), lambda p, i: (jnp.where(p == 1, 0, i), 0))
    return pl.pallas_call(
        _layer1_body,
        grid=(3, NBLK),
        in_specs=[pl.BlockSpec(memory_space=pltpu.SMEM),
                  p2_spec, p02_spec, p0_spec, p0_spec,
                  _vspec((D, D)), _vspec((1, D)), _vspec((1, D)),
                  _vspec((1, D)), _vspec((D, D)), _vspec((1, D)),
                  _vspec((1, D)), _vspec((1, D))],
        out_specs=pl.BlockSpec((BLK, 3 * D),
                               lambda p, i: (jnp.where(p == 2, i, 0), 0)),
        out_shape=jax.ShapeDtypeStruct((N, 3 * D), jnp.float32),
        scratch_shapes=_scratch,
    )(eps2, x, o, a0, a1, Wa, ba2, gm2, bm2, Wb, bb2, g2, b2)


def kernel(x, edge_index, eps0, Wa0, ba0, gm0, bm0, Wb0, bb0, g0, b0,
           eps1, Wa1, ba1, gm1, bm1, Wb1, bb1, g1, b1):
    pad = E_PAD - E
    iot = jnp.arange(pad, dtype=jnp.int32)
    src_p = jnp.concatenate([edge_index[0].astype(jnp.int32), iot % N])
    dst_p = jnp.concatenate([edge_index[1].astype(jnp.int32),
                             N + (iot % (N_PAD - N))])
    src3d = src_p.reshape(NW, STEPS, CHUNK)
    dst3d = dst_p.reshape(NW, STEPS, CHUNK)

    aggp0 = _sc_agg(x, src3d, dst3d)
    o1 = _gin_layer0(x, aggp0[0], aggp0[1], eps0, Wa0, ba0, gm0, bm0,
                     Wb0, bb0, g0, b0)
    aggp1 = _sc_agg(o1, src3d, dst3d)
    return _gin_layer1(x, o1, aggp1[0], aggp1[1], eps1, Wa1, ba1, gm1, bm1,
                       Wb1, bb1, g1, b1)


# speedup vs baseline: 9.4788x; 1.0319x over previous
import functools

import jax
import jax.numpy as jnp
from jax import lax
from jax.experimental import pallas as pl
from jax.experimental.pallas import tpu as pltpu
from jax.experimental.pallas import tpu_sc as plsc

N = 10000
D = 128
E = 320000

CHUNK = 128
NC, NS = 2, 16
NW = NC * NS
STEPS = 80
PSTEPS = 40
E_PAD = NW * STEPS * CHUNK
ROWS_PAD = 640
N_PAD = ROWS_PAD * NS

BLK = 2000
NBLK = N // BLK


def _sc_agg(x, src3d, dst3d):
    mesh = plsc.VectorSubcoreMesh(core_axis_name="c", subcore_axis_name="s")

    @functools.partial(
        pl.kernel,
        out_type=jax.ShapeDtypeStruct((NC, N, D), jnp.float32),
        mesh=mesh,
        scratch_types=[
            pltpu.VMEM((PSTEPS, CHUNK), jnp.int32),
            pltpu.VMEM((PSTEPS, CHUNK), jnp.int32),
            pltpu.VMEM((CHUNK, D), jnp.float32),
            pltpu.VMEM((CHUNK, D), jnp.float32),
            pltpu.VMEM_SHARED((N_PAD, D), jnp.float32),
            pltpu.SemaphoreType.DMA,
            pltpu.SemaphoreType.DMA,
            pltpu.SemaphoreType.DMA,
        ],
    )
    def k(x_hbm, src_hbm, dst_hbm, out_hbm, sidx, didx, bufa, bufb, agg,
          isem, sema, semb):
        cid = lax.axis_index("c")
        sid = lax.axis_index("s")
        wid = sid * NC + cid

        for p in range(STEPS // PSTEPS):
            pltpu.async_copy(src_hbm.at[wid, pl.ds(p * PSTEPS, PSTEPS)],
                             sidx, isem)
            pltpu.async_copy(dst_hbm.at[wid, pl.ds(p * PSTEPS, PSTEPS)],
                             didx, isem)

            if p == 0:
                @pl.loop(0, CHUNK)
                def _(r):
                    @pl.loop(0, D, step=16)
                    def _(c):
                        bufa[r, pl.ds(c, 16)] = jnp.zeros((16,), jnp.float32)

                zbase = sid * ROWS_PAD

                @pl.loop(0, ROWS_PAD // CHUNK)
                def _(t):
                    pltpu.sync_copy(bufa, agg.at[pl.ds(zbase + t * CHUNK,
                                                       CHUNK)])

            pltpu.make_async_copy(src_hbm.at[wid, pl.ds(p * PSTEPS, PSTEPS)],
                                  sidx, isem).wait()
            pltpu.make_async_copy(dst_hbm.at[wid, pl.ds(p * PSTEPS, PSTEPS)],
                                  didx, isem).wait()
            if p == 0:
                plsc.subcore_barrier()

            pltpu.async_copy(x_hbm.at[sidx.at[0]], bufa, sema)

            @pl.loop(0, PSTEPS, step=2)
            def _(t):
                pltpu.make_async_copy(x_hbm.at[sidx.at[t]], bufa, sema).wait()
                pltpu.async_copy(x_hbm.at[sidx.at[t + 1]], bufb, semb)
                pltpu.sync_copy(bufa, agg.at[didx.at[t]], add=True)
                pltpu.make_async_copy(x_hbm.at[sidx.at[t + 1]], bufb,
                                      semb).wait()

                @pl.when(t + 2 < PSTEPS)
                def _():
                    pltpu.async_copy(x_hbm.at[sidx.at[t + 2]], bufa, sema)

                pltpu.sync_copy(bufb, agg.at[didx.at[t + 1]], add=True)

        plsc.subcore_barrier()

        obase = sid * ROWS_PAD

        @pl.when(sid < NS - 1)
        def _():
            pltpu.sync_copy(agg.at[pl.ds(obase, ROWS_PAD)],
                            out_hbm.at[cid, pl.ds(obase, ROWS_PAD)])

        @pl.when(sid == NS - 1)
        def _():
            pltpu.sync_copy(agg.at[pl.ds(obase, N - (NS - 1) * ROWS_PAD)],
                            out_hbm.at[cid, pl.ds(obase, N - (NS - 1) * ROWS_PAD)])

    return k(x, src3d, dst3d)


def _dot(a, b):
    return lax.dot_general(a, b, (((1,), (0,)), ((), ())),
                           preferred_element_type=jnp.float32,
                           precision=lax.Precision.DEFAULT)


def _phases01(p, i, eps_ref, o_ref, a0_ref, a1_ref, Wa_ref, ba_ref, gm_ref,
              bm_ref, Wb_ref, bb_ref, h1_ref, h2_ref, st_ref):
    @pl.when(jnp.logical_and(p == 0, i == 0))
    def _():
        st_ref[...] = jnp.zeros_like(st_ref)

    @pl.when(p == 0)
    def _():
        pre = (1.0 + eps_ref[0, 0]) * o_ref[...] + a0_ref[...] + a1_ref[...]
        h = _dot(pre, Wa_ref[...]) + ba_ref[...]
        h1_ref[pl.ds(i * BLK, BLK), :] = h
        st_ref[0:1, :] += jnp.sum(h, axis=0, keepdims=True)
        st_ref[1:2, :] += jnp.sum(h * h, axis=0, keepdims=True)

    @pl.when(p == 1)
    def _():
        mu = st_ref[0:1, :] * (1.0 / N)
        var = st_ref[1:2, :] * (1.0 / N) - mu * mu
        inv = lax.rsqrt(var + 1e-5)
        hn = (h1_ref[pl.ds(i * BLK, BLK), :] - mu) * (inv * gm_ref[...])
        hn = jnp.maximum(hn + bm_ref[...], 0.0)
        h2 = _dot(hn, Wb_ref[...]) + bb_ref[...]
        h2_ref[pl.ds(i * BLK, BLK), :] = h2
        st_ref[2:3, :] += jnp.sum(h2, axis=0, keepdims=True)
        st_ref[3:4, :] += jnp.sum(h2 * h2, axis=0, keepdims=True)


def _bn2(h2, st_ref, g_ref, b_ref):
    mu = st_ref[2:3, :] * (1.0 / N)
    var = st_ref[3:4, :] * (1.0 / N) - mu * mu
    inv = lax.rsqrt(var + 1e-5)
    return jnp.maximum((h2 - mu) * (inv * g_ref[...]) + b_ref[...], 0.0)


def _layer0_body(eps_ref, o_ref, a0_ref, a1_ref, Wa_ref, ba_ref, gm_ref,
                 bm_ref, Wb_ref, bb_ref, g_ref, b_ref, out_ref,
                 h1_ref, h2_ref, st_ref):
    p, i = pl.program_id(0), pl.program_id(1)
    _phases01(p, i, eps_ref, o_ref, a0_ref, a1_ref, Wa_ref, ba_ref, gm_ref,
              bm_ref, Wb_ref, bb_ref, h1_ref, h2_ref, st_ref)

    @pl.when(p == 2)
    def _():
        out_ref[...] = _bn2(h2_ref[pl.ds(i * BLK, BLK), :], st_ref,
                            g_ref, b_ref)


def _layer1_body(eps_ref, x_ref, o_ref, a0_ref, a1_ref, Wa_ref, ba_ref,
                 gm_ref, bm_ref, Wb_ref, bb_ref, g_ref, b_ref, out_ref,
                 h1_ref, h2_ref, st_ref):
    p, i = pl.program_id(0), pl.program_id(1)
    _phases01(p, i, eps_ref, o_ref, a0_ref, a1_ref, Wa_ref, ba_ref, gm_ref,
              bm_ref, Wb_ref, bb_ref, h1_ref, h2_ref, st_ref)

    @pl.when(p == 2)
    def _():
        out_ref[:, 0:D] = x_ref[...]
        out_ref[:, D:2 * D] = o_ref[...]
        out_ref[:, 2 * D:3 * D] = _bn2(h2_ref[pl.ds(i * BLK, BLK), :],
                                       st_ref, g_ref, b_ref)


def _vspec(shape):
    return pl.BlockSpec(shape, lambda p, i: tuple(0 for _ in shape))


_scratch = [pltpu.VMEM((N, D), jnp.float32),
            pltpu.VMEM((N, D), jnp.float32),
            pltpu.VMEM((8, D), jnp.float32)]


def _prep(eps, vecs):
    return (jnp.reshape(eps, (1, 1)),) + tuple(v.reshape(1, D) for v in vecs)


def _gin_layer0(o, a0, a1, eps, Wa, ba, gm, bm, Wb, bb, g, b):
    eps2, ba2, gm2, bm2, bb2, g2, b2 = _prep(eps, (ba, gm, bm, bb, g, b))
    p0_spec = pl.BlockSpec((BLK, D), lambda p, i: (jnp.where(p == 0, i, 0), 0))
    return pl.pallas_call(
        _layer0_body,
        grid=(3, NBLK),
        in_specs=[pl.BlockSpec(memory_space=pltpu.SMEM),
                  p0_spec, p0_spec, p0_spec,
                  _vspec((D, D)), _vspec((1, D)), _vspec((1, D)),
                  _vspec((1, D)), _vspec((D, D)), _vspec((1, D)),
                  _vspec((1, D)), _vspec((1, D))],
        out_specs=pl.BlockSpec((BLK, D),
                               lambda p, i: (jnp.where(p == 2, i, 0), 0)),
        out_shape=jax.ShapeDtypeStruct((N, D), jnp.float32),
        scratch_shapes=_scratch,
    )(eps2, o, a0, a1, Wa, ba2, gm2, bm2, Wb, bb2, g2, b2)


def _gin_layer1(x, o, a0, a1, eps, Wa, ba, gm, bm, Wb, bb, g, b):
    eps2, ba2, gm2, bm2, bb2, g2, b2 = _prep(eps, (ba, gm, bm, bb, g, b))
    p0_spec = pl.BlockSpec((BLK, D), lambda p, i: (jnp.where(p == 0, i, 0), 0))
    p2_spec = pl.BlockSpec((BLK, D), lambda p, i: (jnp.where(p == 2, i, 0), 0))
    p02_spec = pl.BlockSpec((BLK, D), lambda p, i: (jnp.where(p == 1, 0, i), 0))
    return pl.pallas_call(
        _layer1_body,
        grid=(3, NBLK),
        in_specs=[pl.BlockSpec(memory_space=pltpu.SMEM),
                  p2_spec, p02_spec, p0_spec, p0_spec,
                  _vspec((D, D)), _vspec((1, D)), _vspec((1, D)),
                  _vspec((1, D)), _vspec((D, D)), _vspec((1, D)),
                  _vspec((1, D)), _vspec((1, D))],
        out_specs=pl.BlockSpec((BLK, 3 * D),
                               lambda p, i: (jnp.where(p == 2, i, 0), 0)),
        out_shape=jax.ShapeDtypeStruct((N, 3 * D), jnp.float32),
        scratch_shapes=_scratch,
    )(eps2, x, o, a0, a1, Wa, ba2, gm2, bm2, Wb, bb2, g2, b2)


def kernel(x, edge_index, eps0, Wa0, ba0, gm0, bm0, Wb0, bb0, g0, b0,
           eps1, Wa1, ba1, gm1, bm1, Wb1, bb1, g1, b1):
    pad = E_PAD - E
    iot = jnp.arange(pad, dtype=jnp.int32)
    src_p = jnp.concatenate([edge_index[0].astype(jnp.int32), iot % N])
    dst_p = jnp.concatenate([edge_index[1].astype(jnp.int32),
                             N + (iot % (N_PAD - N))])
    src3d = src_p.reshape(NW, STEPS, CHUNK)
    dst3d = dst_p.reshape(NW, STEPS, CHUNK)

    aggp0 = _sc_agg(x, src3d, dst3d)
    o1 = _gin_layer0(x, aggp0[0], aggp0[1], eps0, Wa0, ba0, gm0, bm0,
                     Wb0, bb0, g0, b0)
    aggp1 = _sc_agg(o1, src3d, dst3d)
    return _gin_layer1(x, o1, aggp1[0], aggp1[1], eps1, Wa1, ba1, gm1, bm1,
                       Wb1, bb1, g1, b1)


# speedup vs baseline: 10.6497x; 1.1235x over previous
import functools

import jax
import jax.numpy as jnp
from jax import lax
from jax.experimental import pallas as pl
from jax.experimental.pallas import tpu as pltpu
from jax.experimental.pallas import tpu_sc as plsc

N = 10000
D = 128
E = 320000

CHUNK = 64
NC, NS = 2, 16
NW = NC * NS
STEPS = 160
PSTEPS = 40
NBUF = 4
E_PAD = NW * STEPS * CHUNK
ROWS_PAD = 640
N_PAD = ROWS_PAD * NS

BLK = 2000
NBLK = N // BLK


def _sc_agg(x, src3d, dst3d):
    mesh = plsc.VectorSubcoreMesh(core_axis_name="c", subcore_axis_name="s")

    @functools.partial(
        pl.kernel,
        out_type=jax.ShapeDtypeStruct((NC, N, D), jnp.float32),
        mesh=mesh,
        scratch_types=[
            pltpu.VMEM((PSTEPS, CHUNK), jnp.int32),
            pltpu.VMEM((PSTEPS, CHUNK), jnp.int32),
        ] + [pltpu.VMEM((CHUNK, D), jnp.float32) for _ in range(NBUF)] + [
            pltpu.VMEM_SHARED((N_PAD, D), jnp.float32),
            pltpu.SemaphoreType.DMA,
        ] + [pltpu.SemaphoreType.DMA for _ in range(2 * NBUF)],
    )
    def k(x_hbm, src_hbm, dst_hbm, out_hbm, sidx, didx, b0, b1, b2, b3, agg,
          isem, g0, g1, g2, g3, s0, s1, s2, s3):
        bufs = (b0, b1, b2, b3)
        gsem = (g0, g1, g2, g3)
        ssem = (s0, s1, s2, s3)
        cid = lax.axis_index("c")
        sid = lax.axis_index("s")
        wid = sid * NC + cid

        for p in range(STEPS // PSTEPS):
            pltpu.async_copy(src_hbm.at[wid, pl.ds(p * PSTEPS, PSTEPS)],
                             sidx, isem)
            pltpu.async_copy(dst_hbm.at[wid, pl.ds(p * PSTEPS, PSTEPS)],
                             didx, isem)

            if p == 0:
                @pl.loop(0, CHUNK)
                def _(r):
                    @pl.loop(0, D, step=16)
                    def _(c):
                        b0[r, pl.ds(c, 16)] = jnp.zeros((16,), jnp.float32)

                zbase = sid * ROWS_PAD

                @pl.loop(0, ROWS_PAD // CHUNK)
                def _(t):
                    pltpu.sync_copy(b0, agg.at[pl.ds(zbase + t * CHUNK,
                                                     CHUNK)])

            pltpu.make_async_copy(src_hbm.at[wid, pl.ds(p * PSTEPS, PSTEPS)],
                                  sidx, isem).wait()
            pltpu.make_async_copy(dst_hbm.at[wid, pl.ds(p * PSTEPS, PSTEPS)],
                                  didx, isem).wait()
            if p == 0:
                plsc.subcore_barrier()

            for b in range(NBUF - 1):
                pltpu.async_copy(x_hbm.at[sidx.at[b]], bufs[b], gsem[b])

            @pl.loop(0, PSTEPS, step=NBUF)
            def _(t):
                for b in range(NBUF):
                    c = t + b
                    nb = (b + NBUF - 1) % NBUF
                    pltpu.make_async_copy(x_hbm.at[sidx.at[c]], bufs[b],
                                          gsem[b]).wait()
                    pltpu.async_copy(bufs[b], agg.at[didx.at[c]], ssem[b],
                                     add=True)

                    @pl.when(c + NBUF - 1 < PSTEPS)
                    def _():
                        @pl.when(c - 1 >= 0)
                        def _():
                            pltpu.make_async_copy(
                                bufs[nb], agg.at[didx.at[c - 1]],
                                ssem[nb]).wait()

                        pltpu.async_copy(x_hbm.at[sidx.at[c + NBUF - 1]],
                                         bufs[nb], gsem[nb])

            for kk in range(NBUF):
                c = PSTEPS - NBUF + kk
                pltpu.make_async_copy(bufs[c % NBUF], agg.at[didx.at[c]],
                                      ssem[c % NBUF]).wait()

        plsc.subcore_barrier()

        obase = sid * ROWS_PAD

        @pl.when(sid < NS - 1)
        def _():
            pltpu.sync_copy(agg.at[pl.ds(obase, ROWS_PAD)],
                            out_hbm.at[cid, pl.ds(obase, ROWS_PAD)])

        @pl.when(sid == NS - 1)
        def _():
            pltpu.sync_copy(agg.at[pl.ds(obase, N - (NS - 1) * ROWS_PAD)],
                            out_hbm.at[cid, pl.ds(obase, N - (NS - 1) * ROWS_PAD)])

    return k(x, src3d, dst3d)


def _dot(a, b):
    return lax.dot_general(a, b, (((1,), (0,)), ((), ())),
                           preferred_element_type=jnp.float32,
                           precision=lax.Precision.DEFAULT)


def _phases01(p, i, eps_ref, o_ref, a0_ref, a1_ref, Wa_ref, ba_ref, gm_ref,
              bm_ref, Wb_ref, bb_ref, h1_ref, h2_ref, st_ref):
    @pl.when(jnp.logical_and(p == 0, i == 0))
    def _():
        st_ref[...] = jnp.zeros_like(st_ref)

    @pl.when(p == 0)
    def _():
        pre = (1.0 + eps_ref[0, 0]) * o_ref[...] + a0_ref[...] + a1_ref[...]
        h = _dot(pre, Wa_ref[...]) + ba_ref[...]
        h1_ref[pl.ds(i * BLK, BLK), :] = h
        st_ref[0:1, :] += jnp.sum(h, axis=0, keepdims=True)
        st_ref[1:2, :] += jnp.sum(h * h, axis=0, keepdims=True)

    @pl.when(p == 1)
    def _():
        mu = st_ref[0:1, :] * (1.0 / N)
        var = st_ref[1:2, :] * (1.0 / N) - mu * mu
        inv = lax.rsqrt(var + 1e-5)
        hn = (h1_ref[pl.ds(i * BLK, BLK), :] - mu) * (inv * gm_ref[...])
        hn = jnp.maximum(hn + bm_ref[...], 0.0)
        h2 = _dot(hn, Wb_ref[...]) + bb_ref[...]
        h2_ref[pl.ds(i * BLK, BLK), :] = h2
        st_ref[2:3, :] += jnp.sum(h2, axis=0, keepdims=True)
        st_ref[3:4, :] += jnp.sum(h2 * h2, axis=0, keepdims=True)


def _bn2(h2, st_ref, g_ref, b_ref):
    mu = st_ref[2:3, :] * (1.0 / N)
    var = st_ref[3:4, :] * (1.0 / N) - mu * mu
    inv = lax.rsqrt(var + 1e-5)
    return jnp.maximum((h2 - mu) * (inv * g_ref[...]) + b_ref[...], 0.0)


def _layer0_body(eps_ref, o_ref, a0_ref, a1_ref, Wa_ref, ba_ref, gm_ref,
                 bm_ref, Wb_ref, bb_ref, g_ref, b_ref, out_ref,
                 h1_ref, h2_ref, st_ref):
    p, i = pl.program_id(0), pl.program_id(1)
    _phases01(p, i, eps_ref, o_ref, a0_ref, a1_ref, Wa_ref, ba_ref, gm_ref,
              bm_ref, Wb_ref, bb_ref, h1_ref, h2_ref, st_ref)

    @pl.when(p == 2)
    def _():
        out_ref[...] = _bn2(h2_ref[pl.ds(i * BLK, BLK), :], st_ref,
                            g_ref, b_ref)


def _layer1_body(eps_ref, x_ref, o_ref, a0_ref, a1_ref, Wa_ref, ba_ref,
                 gm_ref, bm_ref, Wb_ref, bb_ref, g_ref, b_ref, out_ref,
                 h1_ref, h2_ref, st_ref):
    p, i = pl.program_id(0), pl.program_id(1)
    _phases01(p, i, eps_ref, o_ref, a0_ref, a1_ref, Wa_ref, ba_ref, gm_ref,
              bm_ref, Wb_ref, bb_ref, h1_ref, h2_ref, st_ref)

    @pl.when(p == 2)
    def _():
        out_ref[:, 0:D] = x_ref[...]
        out_ref[:, D:2 * D] = o_ref[...]
        out_ref[:, 2 * D:3 * D] = _bn2(h2_ref[pl.ds(i * BLK, BLK), :],
                                       st_ref, g_ref, b_ref)


def _vspec(shape):
    return pl.BlockSpec(shape, lambda p, i: tuple(0 for _ in shape))


_scratch = [pltpu.VMEM((N, D), jnp.float32),
            pltpu.VMEM((N, D), jnp.float32),
            pltpu.VMEM((8, D), jnp.float32)]


def _prep(eps, vecs):
    return (jnp.reshape(eps, (1, 1)),) + tuple(v.reshape(1, D) for v in vecs)


def _gin_layer0(o, a0, a1, eps, Wa, ba, gm, bm, Wb, bb, g, b):
    eps2, ba2, gm2, bm2, bb2, g2, b2 = _prep(eps, (ba, gm, bm, bb, g, b))
    p0_spec = pl.BlockSpec((BLK, D), lambda p, i: (jnp.where(p == 0, i, 0), 0))
    return pl.pallas_call(
        _layer0_body,
        grid=(3, NBLK),
        in_specs=[pl.BlockSpec(memory_space=pltpu.SMEM),
                  p0_spec, p0_spec, p0_spec,
                  _vspec((D, D)), _vspec((1, D)), _vspec((1, D)),
                  _vspec((1, D)), _vspec((D, D)), _vspec((1, D)),
                  _vspec((1, D)), _vspec((1, D))],
        out_specs=pl.BlockSpec((BLK, D),
                               lambda p, i: (jnp.where(p == 2, i, 0), 0)),
        out_shape=jax.ShapeDtypeStruct((N, D), jnp.float32),
        scratch_shapes=_scratch,
    )(eps2, o, a0, a1, Wa, ba2, gm2, bm2, Wb, bb2, g2, b2)


def _gin_layer1(x, o, a0, a1, eps, Wa, ba, gm, bm, Wb, bb, g, b):
    eps2, ba2, gm2, bm2, bb2, g2, b2 = _prep(eps, (ba, gm, bm, bb, g, b))
    p0_spec = pl.BlockSpec((BLK, D), lambda p, i: (jnp.where(p == 0, i, 0), 0))
    p2_spec = pl.BlockSpec((BLK, D), lambda p, i: (jnp.where(p == 2, i, 0), 0))
    p02_spec = pl.BlockSpec((BLK, D), lambda p, i: (jnp.where(p == 1, 0, i), 0))
    return pl.pallas_call(
        _layer1_body,
        grid=(3, NBLK),
        in_specs=[pl.BlockSpec(memory_space=pltpu.SMEM),
                  p2_spec, p02_spec, p0_spec, p0_spec,
                  _vspec((D, D)), _vspec((1, D)), _vspec((1, D)),
                  _vspec((1, D)), _vspec((D, D)), _vspec((1, D)),
                  _vspec((1, D)), _vspec((1, D))],
        out_specs=pl.BlockSpec((BLK, 3 * D),
                               lambda p, i: (jnp.where(p == 2, i, 0), 0)),
        out_shape=jax.ShapeDtypeStruct((N, 3 * D), jnp.float32),
        scratch_shapes=_scratch,
    )(eps2, x, o, a0, a1, Wa, ba2, gm2, bm2, Wb, bb2, g2, b2)


def kernel(x, edge_index, eps0, Wa0, ba0, gm0, bm0, Wb0, bb0, g0, b0,
           eps1, Wa1, ba1, gm1, bm1, Wb1, bb1, g1, b1):
    pad = E_PAD - E
    iot = jnp.arange(pad, dtype=jnp.int32)
    src_p = jnp.concatenate([edge_index[0].astype(jnp.int32), iot % N])
    dst_p = jnp.concatenate([edge_index[1].astype(jnp.int32),
                             N + (iot % (N_PAD - N))])
    src3d = src_p.reshape(NW, STEPS, CHUNK)
    dst3d = dst_p.reshape(NW, STEPS, CHUNK)

    aggp0 = _sc_agg(x, src3d, dst3d)
    o1 = _gin_layer0(x, aggp0[0], aggp0[1], eps0, Wa0, ba0, gm0, bm0,
                     Wb0, bb0, g0, b0)
    aggp1 = _sc_agg(o1, src3d, dst3d)
    return _gin_layer1(x, o1, aggp1[0], aggp1[1], eps1, Wa1, ba1, gm1, bm1,
                       Wb1, bb1, g1, b1)


# speedup vs baseline: 10.7473x; 1.0092x over previous
import functools

import jax
import jax.numpy as jnp
from jax import lax
from jax.experimental import pallas as pl
from jax.experimental.pallas import tpu as pltpu
from jax.experimental.pallas import tpu_sc as plsc

N = 10000
D = 128
E = 320000

CHUNK = 64
NC, NS = 2, 16
NW = NC * NS
STEPS = 160
PSTEPS = 40
NBUF = 4
E_PAD = NW * STEPS * CHUNK
ROWS_PAD = 640
N_PAD = ROWS_PAD * NS

BLK = 5000
NBLK = N // BLK


def _sc_agg(x, src3d, dst3d):
    mesh = plsc.VectorSubcoreMesh(core_axis_name="c", subcore_axis_name="s")

    @functools.partial(
        pl.kernel,
        out_type=jax.ShapeDtypeStruct((NC, N, D), jnp.float32),
        mesh=mesh,
        scratch_types=[
            pltpu.VMEM((PSTEPS, CHUNK), jnp.int32),
            pltpu.VMEM((PSTEPS, CHUNK), jnp.int32),
        ] + [pltpu.VMEM((CHUNK, D), jnp.float32) for _ in range(NBUF)] + [
            pltpu.VMEM_SHARED((N_PAD, D), jnp.float32),
            pltpu.SemaphoreType.DMA,
        ] + [pltpu.SemaphoreType.DMA for _ in range(2 * NBUF)],
    )
    def k(x_hbm, src_hbm, dst_hbm, out_hbm, sidx, didx, b0, b1, b2, b3, agg,
          isem, g0, g1, g2, g3, s0, s1, s2, s3):
        bufs = (b0, b1, b2, b3)
        gsem = (g0, g1, g2, g3)
        ssem = (s0, s1, s2, s3)
        cid = lax.axis_index("c")
        sid = lax.axis_index("s")
        wid = sid * NC + cid

        for p in range(STEPS // PSTEPS):
            pltpu.async_copy(src_hbm.at[wid, pl.ds(p * PSTEPS, PSTEPS)],
                             sidx, isem)
            pltpu.async_copy(dst_hbm.at[wid, pl.ds(p * PSTEPS, PSTEPS)],
                             didx, isem)

            if p == 0:
                @pl.loop(0, CHUNK)
                def _(r):
                    @pl.loop(0, D, step=16)
                    def _(c):
                        b0[r, pl.ds(c, 16)] = jnp.zeros((16,), jnp.float32)

                zbase = sid * ROWS_PAD

                @pl.loop(0, ROWS_PAD // CHUNK)
                def _(t):
                    pltpu.sync_copy(b0, agg.at[pl.ds(zbase + t * CHUNK,
                                                     CHUNK)])

            pltpu.make_async_copy(src_hbm.at[wid, pl.ds(p * PSTEPS, PSTEPS)],
                                  sidx, isem).wait()
            pltpu.make_async_copy(dst_hbm.at[wid, pl.ds(p * PSTEPS, PSTEPS)],
                                  didx, isem).wait()
            if p == 0:
                plsc.subcore_barrier()

            for b in range(NBUF - 1):
                pltpu.async_copy(x_hbm.at[sidx.at[b]], bufs[b], gsem[b])

            @pl.loop(0, PSTEPS, step=NBUF)
            def _(t):
                for b in range(NBUF):
                    c = t + b
                    nb = (b + NBUF - 1) % NBUF
                    pltpu.make_async_copy(x_hbm.at[sidx.at[c]], bufs[b],
                                          gsem[b]).wait()
                    pltpu.async_copy(bufs[b], agg.at[didx.at[c]], ssem[b],
                                     add=True)

                    @pl.when(c + NBUF - 1 < PSTEPS)
                    def _():
                        @pl.when(c - 1 >= 0)
                        def _():
                            pltpu.make_async_copy(
                                bufs[nb], agg.at[didx.at[c - 1]],
                                ssem[nb]).wait()

                        pltpu.async_copy(x_hbm.at[sidx.at[c + NBUF - 1]],
                                         bufs[nb], gsem[nb])

            for kk in range(NBUF):
                c = PSTEPS - NBUF + kk
                pltpu.make_async_copy(bufs[c % NBUF], agg.at[didx.at[c]],
                                      ssem[c % NBUF]).wait()

        plsc.subcore_barrier()

        obase = sid * ROWS_PAD

        @pl.when(sid < NS - 1)
        def _():
            pltpu.sync_copy(agg.at[pl.ds(obase, ROWS_PAD)],
                            out_hbm.at[cid, pl.ds(obase, ROWS_PAD)])

        @pl.when(sid == NS - 1)
        def _():
            pltpu.sync_copy(agg.at[pl.ds(obase, N - (NS - 1) * ROWS_PAD)],
                            out_hbm.at[cid, pl.ds(obase, N - (NS - 1) * ROWS_PAD)])

    return k(x, src3d, dst3d)


def _dot(a, b):
    return lax.dot_general(a, b, (((1,), (0,)), ((), ())),
                           preferred_element_type=jnp.float32,
                           precision=lax.Precision.DEFAULT)


def _phases01(p, i, eps_ref, o_ref, a0_ref, a1_ref, Wa_ref, ba_ref, gm_ref,
              bm_ref, Wb_ref, bb_ref, h1_ref, h2_ref, st_ref):
    @pl.when(jnp.logical_and(p == 0, i == 0))
    def _():
        st_ref[...] = jnp.zeros_like(st_ref)

    @pl.when(p == 0)
    def _():
        pre = (1.0 + eps_ref[0, 0]) * o_ref[...] + a0_ref[...] + a1_ref[...]
        h = _dot(pre, Wa_ref[...]) + ba_ref[...]
        h1_ref[pl.ds(i * BLK, BLK), :] = h
        st_ref[0:1, :] += jnp.sum(h, axis=0, keepdims=True)
        st_ref[1:2, :] += jnp.sum(h * h, axis=0, keepdims=True)

    @pl.when(p == 1)
    def _():
        mu = st_ref[0:1, :] * (1.0 / N)
        var = st_ref[1:2, :] * (1.0 / N) - mu * mu
        inv = lax.rsqrt(var + 1e-5)
        hn = (h1_ref[pl.ds(i * BLK, BLK), :] - mu) * (inv * gm_ref[...])
        hn = jnp.maximum(hn + bm_ref[...], 0.0)
        h2 = _dot(hn, Wb_ref[...]) + bb_ref[...]
        h2_ref[pl.ds(i * BLK, BLK), :] = h2
        st_ref[2:3, :] += jnp.sum(h2, axis=0, keepdims=True)
        st_ref[3:4, :] += jnp.sum(h2 * h2, axis=0, keepdims=True)


def _bn2(h2, st_ref, g_ref, b_ref):
    mu = st_ref[2:3, :] * (1.0 / N)
    var = st_ref[3:4, :] * (1.0 / N) - mu * mu
    inv = lax.rsqrt(var + 1e-5)
    return jnp.maximum((h2 - mu) * (inv * g_ref[...]) + b_ref[...], 0.0)


def _layer0_body(eps_ref, o_ref, a0_ref, a1_ref, Wa_ref, ba_ref, gm_ref,
                 bm_ref, Wb_ref, bb_ref, g_ref, b_ref, out_ref,
                 h1_ref, h2_ref, st_ref):
    p, i = pl.program_id(0), pl.program_id(1)
    _phases01(p, i, eps_ref, o_ref, a0_ref, a1_ref, Wa_ref, ba_ref, gm_ref,
              bm_ref, Wb_ref, bb_ref, h1_ref, h2_ref, st_ref)

    @pl.when(p == 2)
    def _():
        out_ref[...] = _bn2(h2_ref[pl.ds(i * BLK, BLK), :], st_ref,
                            g_ref, b_ref)


def _layer1_body(eps_ref, x_ref, o_ref, a0_ref, a1_ref, Wa_ref, ba_ref,
                 gm_ref, bm_ref, Wb_ref, bb_ref, g_ref, b_ref, out_ref,
                 h1_ref, h2_ref, st_ref):
    p, i = pl.program_id(0), pl.program_id(1)
    _phases01(p, i, eps_ref, o_ref, a0_ref, a1_ref, Wa_ref, ba_ref, gm_ref,
              bm_ref, Wb_ref, bb_ref, h1_ref, h2_ref, st_ref)

    @pl.when(p == 2)
    def _():
        out_ref[:, 0:D] = x_ref[...]
        out_ref[:, D:2 * D] = o_ref[...]
        out_ref[:, 2 * D:3 * D] = _bn2(h2_ref[pl.ds(i * BLK, BLK), :],
                                       st_ref, g_ref, b_ref)


def _vspec(shape):
    return pl.BlockSpec(shape, lambda p, i: tuple(0 for _ in shape))


_scratch = [pltpu.VMEM((N, D), jnp.float32),
            pltpu.VMEM((N, D), jnp.float32),
            pltpu.VMEM((8, D), jnp.float32)]


def _prep(eps, vecs):
    return (jnp.reshape(eps, (1, 1)),) + tuple(v.reshape(1, D) for v in vecs)


def _gin_layer0(o, a0, a1, eps, Wa, ba, gm, bm, Wb, bb, g, b):
    eps2, ba2, gm2, bm2, bb2, g2, b2 = _prep(eps, (ba, gm, bm, bb, g, b))
    p0_spec = pl.BlockSpec((BLK, D), lambda p, i: (jnp.where(p == 0, i, 0), 0))
    return pl.pallas_call(
        _layer0_body,
        grid=(3, NBLK),
        in_specs=[pl.BlockSpec(memory_space=pltpu.SMEM),
                  p0_spec, p0_spec, p0_spec,
                  _vspec((D, D)), _vspec((1, D)), _vspec((1, D)),
                  _vspec((1, D)), _vspec((D, D)), _vspec((1, D)),
                  _vspec((1, D)), _vspec((1, D))],
        out_specs=pl.BlockSpec((BLK, D),
                               lambda p, i: (jnp.where(p == 2, i, 0), 0)),
        out_shape=jax.ShapeDtypeStruct((N, D), jnp.float32),
        scratch_shapes=_scratch,
    )(eps2, o, a0, a1, Wa, ba2, gm2, bm2, Wb, bb2, g2, b2)


def _gin_layer1(x, o, a0, a1, eps, Wa, ba, gm, bm, Wb, bb, g, b):
    eps2, ba2, gm2, bm2, bb2, g2, b2 = _prep(eps, (ba, gm, bm, bb, g, b))
    p0_spec = pl.BlockSpec((BLK, D), lambda p, i: (jnp.where(p == 0, i, 0), 0))
    p2_spec = pl.BlockSpec((BLK, D), lambda p, i: (jnp.where(p == 2, i, 0), 0))
    p02_spec = pl.BlockSpec((BLK, D), lambda p, i: (jnp.where(p == 1, 0, i), 0))
    return pl.pallas_call(
        _layer1_body,
        grid=(3, NBLK),
        in_specs=[pl.BlockSpec(memory_space=pltpu.SMEM),
                  p2_spec, p02_spec, p0_spec, p0_spec,
                  _vspec((D, D)), _vspec((1, D)), _vspec((1, D)),
                  _vspec((1, D)), _vspec((D, D)), _vspec((1, D)),
                  _vspec((1, D)), _vspec((1, D))],
        out_specs=pl.BlockSpec((BLK, 3 * D),
                               lambda p, i: (jnp.where(p == 2, i, 0), 0)),
        out_shape=jax.ShapeDtypeStruct((N, 3 * D), jnp.float32),
        scratch_shapes=_scratch,
    )(eps2, x, o, a0, a1, Wa, ba2, gm2, bm2, Wb, bb2, g2, b2)


def kernel(x, edge_index, eps0, Wa0, ba0, gm0, bm0, Wb0, bb0, g0, b0,
           eps1, Wa1, ba1, gm1, bm1, Wb1, bb1, g1, b1):
    pad = E_PAD - E
    iot = jnp.arange(pad, dtype=jnp.int32)
    src_p = jnp.concatenate([edge_index[0].astype(jnp.int32), iot % N])
    dst_p = jnp.concatenate([edge_index[1].astype(jnp.int32),
                             N + (iot % (N_PAD - N))])
    src3d = src_p.reshape(NW, STEPS, CHUNK)
    dst3d = dst_p.reshape(NW, STEPS, CHUNK)

    aggp0 = _sc_agg(x, src3d, dst3d)
    o1 = _gin_layer0(x, aggp0[0], aggp0[1], eps0, Wa0, ba0, gm0, bm0,
                     Wb0, bb0, g0, b0)
    aggp1 = _sc_agg(o1, src3d, dst3d)
    return _gin_layer1(x, o1, aggp1[0], aggp1[1], eps1, Wa1, ba1, gm1, bm1,
                       Wb1, bb1, g1, b1)


# speedup vs baseline: 10.8663x; 1.0111x over previous
import functools

import jax
import jax.numpy as jnp
from jax import lax
from jax.experimental import pallas as pl
from jax.experimental.pallas import tpu as pltpu
from jax.experimental.pallas import tpu_sc as plsc

N = 10000
D = 128
E = 320000

CHUNK = 64
NC, NS = 2, 16
NW = NC * NS
STEPS = 160
PSTEPS = 40
NBUF = 4
E_PAD = NW * STEPS * CHUNK
ROWS_PAD = 640
N_PAD = ROWS_PAD * NS

BLK = 5000
NBLK = N // BLK


def _sc_agg(x, src3d, dst3d):
    mesh = plsc.VectorSubcoreMesh(core_axis_name="c", subcore_axis_name="s")

    @functools.partial(
        pl.kernel,
        out_type=jax.ShapeDtypeStruct((NC, N, D), jnp.float32),
        mesh=mesh,
        scratch_types=[
            pltpu.VMEM((PSTEPS, CHUNK), jnp.int32),
            pltpu.VMEM((PSTEPS, CHUNK), jnp.int32),
        ] + [pltpu.VMEM((CHUNK, D), jnp.float32) for _ in range(NBUF)] + [
            pltpu.VMEM_SHARED((N_PAD, D), jnp.float32),
            pltpu.SemaphoreType.DMA,
        ] + [pltpu.SemaphoreType.DMA for _ in range(2 * NBUF)],
    )
    def k(x_hbm, src_hbm, dst_hbm, out_hbm, sidx, didx, b0, b1, b2, b3, agg,
          isem, g0, g1, g2, g3, s0, s1, s2, s3):
        bufs = (b0, b1, b2, b3)
        gsem = (g0, g1, g2, g3)
        ssem = (s0, s1, s2, s3)
        cid = lax.axis_index("c")
        sid = lax.axis_index("s")
        wid = sid * NC + cid

        for p in range(STEPS // PSTEPS):
            pltpu.async_copy(src_hbm.at[wid, pl.ds(p * PSTEPS, PSTEPS)],
                             sidx, isem)
            pltpu.async_copy(dst_hbm.at[wid, pl.ds(p * PSTEPS, PSTEPS)],
                             didx, isem)

            if p == 0:
                @pl.loop(0, CHUNK)
                def _(r):
                    @pl.loop(0, D, step=16)
                    def _(c):
                        b0[r, pl.ds(c, 16)] = jnp.zeros((16,), jnp.float32)

                zbase = sid * ROWS_PAD

                @pl.loop(0, ROWS_PAD // CHUNK)
                def _(t):
                    pltpu.sync_copy(b0, agg.at[pl.ds(zbase + t * CHUNK,
                                                     CHUNK)])

            pltpu.make_async_copy(src_hbm.at[wid, pl.ds(p * PSTEPS, PSTEPS)],
                                  sidx, isem).wait()
            pltpu.make_async_copy(dst_hbm.at[wid, pl.ds(p * PSTEPS, PSTEPS)],
                                  didx, isem).wait()
            if p == 0:
                plsc.subcore_barrier()

            for b in range(NBUF - 1):
                pltpu.async_copy(x_hbm.at[sidx.at[b]], bufs[b], gsem[b])

            @pl.loop(0, PSTEPS, step=NBUF)
            def _(t):
                for b in range(NBUF):
                    c = t + b
                    nb = (b + NBUF - 1) % NBUF
                    pltpu.make_async_copy(x_hbm.at[sidx.at[c]], bufs[b],
                                          gsem[b]).wait()
                    pltpu.async_copy(bufs[b], agg.at[didx.at[c]], ssem[b],
                                     add=True)

                    @pl.when(c + NBUF - 1 < PSTEPS)
                    def _():
                        @pl.when(c - 1 >= 0)
                        def _():
                            pltpu.make_async_copy(
                                bufs[nb], agg.at[didx.at[c - 1]],
                                ssem[nb]).wait()

                        pltpu.async_copy(x_hbm.at[sidx.at[c + NBUF - 1]],
                                         bufs[nb], gsem[nb])

            for kk in range(NBUF):
                c = PSTEPS - NBUF + kk
                pltpu.make_async_copy(bufs[c % NBUF], agg.at[didx.at[c]],
                                      ssem[c % NBUF]).wait()

        plsc.subcore_barrier()

        obase = sid * ROWS_PAD

        @pl.when(sid < NS - 1)
        def _():
            pltpu.sync_copy(agg.at[pl.ds(obase, ROWS_PAD)],
                            out_hbm.at[cid, pl.ds(obase, ROWS_PAD)])

        @pl.when(sid == NS - 1)
        def _():
            pltpu.sync_copy(agg.at[pl.ds(obase, N - (NS - 1) * ROWS_PAD)],
                            out_hbm.at[cid, pl.ds(obase, N - (NS - 1) * ROWS_PAD)])

    return k(x, src3d, dst3d)


def _dot(a, b):
    return lax.dot_general(a, b, (((1,), (0,)), ((), ())),
                           preferred_element_type=jnp.float32,
                           precision=lax.Precision.DEFAULT)


def _phases01(p, i, eps_ref, o_ref, a0_ref, a1_ref, Wa_ref, ba_ref, gm_ref,
              bm_ref, Wb_ref, bb_ref, h1_ref, h2_ref, st_ref):
    @pl.when(jnp.logical_and(p == 0, i == 0))
    def _():
        st_ref[...] = jnp.zeros_like(st_ref)

    @pl.when(p == 0)
    def _():
        pre = (1.0 + eps_ref[0, 0]) * o_ref[...] + a0_ref[...] + a1_ref[...]
        h = _dot(pre, Wa_ref[...]) + ba_ref[...]
        h1_ref[pl.ds(i * BLK, BLK), :] = h
        st_ref[0:1, :] += jnp.sum(h, axis=0, keepdims=True)
        st_ref[1:2, :] += jnp.sum(h * h, axis=0, keepdims=True)

    @pl.when(p == 1)
    def _():
        mu = st_ref[0:1, :] * (1.0 / N)
        var = st_ref[1:2, :] * (1.0 / N) - mu * mu
        inv = lax.rsqrt(var + 1e-5)
        hn = (h1_ref[pl.ds(i * BLK, BLK), :] - mu) * (inv * gm_ref[...])
        hn = jnp.maximum(hn + bm_ref[...], 0.0)
        h2 = _dot(hn, Wb_ref[...]) + bb_ref[...]
        h2_ref[pl.ds(i * BLK, BLK), :] = h2
        st_ref[2:3, :] += jnp.sum(h2, axis=0, keepdims=True)
        st_ref[3:4, :] += jnp.sum(h2 * h2, axis=0, keepdims=True)


def _bn2(h2, st_ref, g_ref, b_ref):
    mu = st_ref[2:3, :] * (1.0 / N)
    var = st_ref[3:4, :] * (1.0 / N) - mu * mu
    inv = lax.rsqrt(var + 1e-5)
    return jnp.maximum((h2 - mu) * (inv * g_ref[...]) + b_ref[...], 0.0)


def _layer0_body(eps_ref, o_ref, a0_ref, a1_ref, Wa_ref, ba_ref, gm_ref,
                 bm_ref, Wb_ref, bb_ref, g_ref, b_ref, out_ref,
                 h1_ref, h2_ref, st_ref):
    p, i = pl.program_id(0), pl.program_id(1)
    _phases01(p, i, eps_ref, o_ref, a0_ref, a1_ref, Wa_ref, ba_ref, gm_ref,
              bm_ref, Wb_ref, bb_ref, h1_ref, h2_ref, st_ref)

    @pl.when(p == 2)
    def _():
        out_ref[...] = _bn2(h2_ref[pl.ds(i * BLK, BLK), :], st_ref,
                            g_ref, b_ref)


def _layer1_body(eps_ref, base_ref, o_ref, a0_ref, a1_ref, Wa_ref, ba_ref,
                 gm_ref, bm_ref, Wb_ref, bb_ref, g_ref, b_ref, out_ref,
                 h1_ref, h2_ref, st_ref):
    del base_ref
    p, i = pl.program_id(0), pl.program_id(1)
    _phases01(p, i, eps_ref, o_ref, a0_ref, a1_ref, Wa_ref, ba_ref, gm_ref,
              bm_ref, Wb_ref, bb_ref, h1_ref, h2_ref, st_ref)

    @pl.when(p == 2)
    def _():
        out_ref[...] = _bn2(h2_ref[pl.ds(i * BLK, BLK), :], st_ref,
                            g_ref, b_ref)


def _concat01_body(x_ref, o_ref, out_ref):
    j = pl.program_id(0)

    @pl.when(j == 0)
    def _():
        out_ref[...] = x_ref[...]

    @pl.when(j == 1)
    def _():
        out_ref[...] = o_ref[...]


def _concat01(x, o1):
    return pl.pallas_call(
        _concat01_body,
        grid=(2, NBLK),
        in_specs=[
            pl.BlockSpec((BLK, D), lambda j, i: (jnp.where(j == 0, i, 0), 0)),
            pl.BlockSpec((BLK, D), lambda j, i: (jnp.where(j == 1, i, 0), 0)),
        ],
        out_specs=pl.BlockSpec((BLK, D), lambda j, i: (i, j)),
        out_shape=jax.ShapeDtypeStruct((N, 3 * D), jnp.float32),
    )(x, o1)


def _vspec(shape):
    return pl.BlockSpec(shape, lambda p, i: tuple(0 for _ in shape))


_scratch = [pltpu.VMEM((N, D), jnp.float32),
            pltpu.VMEM((N, D), jnp.float32),
            pltpu.VMEM((8, D), jnp.float32)]


def _prep(eps, vecs):
    return (jnp.reshape(eps, (1, 1)),) + tuple(v.reshape(1, D) for v in vecs)


def _gin_layer0(o, a0, a1, eps, Wa, ba, gm, bm, Wb, bb, g, b):
    eps2, ba2, gm2, bm2, bb2, g2, b2 = _prep(eps, (ba, gm, bm, bb, g, b))
    p0_spec = pl.BlockSpec((BLK, D), lambda p, i: (jnp.where(p == 0, i, 0), 0))
    return pl.pallas_call(
        _layer0_body,
        grid=(3, NBLK),
        in_specs=[pl.BlockSpec(memory_space=pltpu.SMEM),
                  p0_spec, p0_spec, p0_spec,
                  _vspec((D, D)), _vspec((1, D)), _vspec((1, D)),
                  _vspec((1, D)), _vspec((D, D)), _vspec((1, D)),
                  _vspec((1, D)), _vspec((1, D))],
        out_specs=pl.BlockSpec((BLK, D),
                               lambda p, i: (jnp.where(p == 2, i, 0), 0)),
        out_shape=jax.ShapeDtypeStruct((N, D), jnp.float32),
        scratch_shapes=_scratch,
    )(eps2, o, a0, a1, Wa, ba2, gm2, bm2, Wb, bb2, g2, b2)


def _gin_layer1(base, o, a0, a1, eps, Wa, ba, gm, bm, Wb, bb, g, b):
    eps2, ba2, gm2, bm2, bb2, g2, b2 = _prep(eps, (ba, gm, bm, bb, g, b))
    p0_spec = pl.BlockSpec((BLK, D), lambda p, i: (jnp.where(p == 0, i, 0), 0))
    return pl.pallas_call(
        _layer1_body,
        grid=(3, NBLK),
        in_specs=[pl.BlockSpec(memory_space=pltpu.SMEM),
                  pl.BlockSpec((8, D), lambda p, i: (0, 0)),
                  p0_spec, p0_spec, p0_spec,
                  _vspec((D, D)), _vspec((1, D)), _vspec((1, D)),
                  _vspec((1, D)), _vspec((D, D)), _vspec((1, D)),
                  _vspec((1, D)), _vspec((1, D))],
        out_specs=pl.BlockSpec((BLK, D),
                               lambda p, i: (jnp.where(p == 2, i, 0), 2)),
        out_shape=jax.ShapeDtypeStruct((N, 3 * D), jnp.float32),
        input_output_aliases={1: 0},
        scratch_shapes=_scratch,
    )(eps2, base, o, a0, a1, Wa, ba2, gm2, bm2, Wb, bb2, g2, b2)


def kernel(x, edge_index, eps0, Wa0, ba0, gm0, bm0, Wb0, bb0, g0, b0,
           eps1, Wa1, ba1, gm1, bm1, Wb1, bb1, g1, b1):
    pad = E_PAD - E
    iot = jnp.arange(pad, dtype=jnp.int32)
    src_p = jnp.concatenate([edge_index[0].astype(jnp.int32), iot % N])
    dst_p = jnp.concatenate([edge_index[1].astype(jnp.int32),
                             N + (iot % (N_PAD - N))])
    src3d = src_p.reshape(NW, STEPS, CHUNK)
    dst3d = dst_p.reshape(NW, STEPS, CHUNK)

    aggp0 = _sc_agg(x, src3d, dst3d)
    o1 = _gin_layer0(x, aggp0[0], aggp0[1], eps0, Wa0, ba0, gm0, bm0,
                     Wb0, bb0, g0, b0)
    aggp1 = _sc_agg(o1, src3d, dst3d)
    base = _concat01(x, o1)
    return _gin_layer1(base, o1, aggp1[0], aggp1[1], eps1, Wa1, ba1, gm1,
                       bm1, Wb1, bb1, g1, b1)


# speedup vs baseline: 11.6976x; 1.0765x over previous
import functools

import jax
import jax.numpy as jnp
import numpy as np
from jax import lax
from jax.experimental import pallas as pl
from jax.experimental.pallas import tpu as pltpu
from jax.experimental.pallas import tpu_sc as plsc

N = 10000
D = 128
E = 320000

CHUNK = 64
NC, NS = 2, 16
NW = NC * NS
STEPS = 160
PSTEPS = 40
NBUF = 4
E_PAD = NW * STEPS * CHUNK
ROWS_PAD = 640
N_PAD = ROWS_PAD * NS

BLK = 5000
NBLK = N // BLK


_PAD_EDGES = np.stack([
    np.arange(E_PAD - E, dtype=np.int32) % N,
    N + np.arange(E_PAD - E, dtype=np.int32) % (N_PAD - N),
]).astype(np.int32)


def _sc_agg(x, e_all):
    mesh = plsc.VectorSubcoreMesh(core_axis_name="c", subcore_axis_name="s")

    @functools.partial(
        pl.kernel,
        out_type=jax.ShapeDtypeStruct((NC, N, D), jnp.float32),
        mesh=mesh,
        scratch_types=[
            pltpu.VMEM((PSTEPS, CHUNK), jnp.int32),
            pltpu.VMEM((PSTEPS, CHUNK), jnp.int32),
        ] + [pltpu.VMEM((CHUNK, D), jnp.float32) for _ in range(NBUF)] + [
            pltpu.VMEM_SHARED((N_PAD, D), jnp.float32),
            pltpu.SemaphoreType.DMA,
        ] + [pltpu.SemaphoreType.DMA for _ in range(2 * NBUF)],
    )
    def k(x_hbm, e_hbm, out_hbm, sidx, didx, b0, b1, b2, b3, agg,
          isem, g0, g1, g2, g3, s0, s1, s2, s3):
        bufs = (b0, b1, b2, b3)
        gsem = (g0, g1, g2, g3)
        ssem = (s0, s1, s2, s3)
        cid = lax.axis_index("c")
        sid = lax.axis_index("s")
        wid = sid * NC + cid

        for p in range(STEPS // PSTEPS):
            pltpu.async_copy(e_hbm.at[0, wid, pl.ds(p * PSTEPS, PSTEPS)],
                             sidx, isem)
            pltpu.async_copy(e_hbm.at[1, wid, pl.ds(p * PSTEPS, PSTEPS)],
                             didx, isem)

            if p == 0:
                @pl.loop(0, CHUNK)
                def _(r):
                    @pl.loop(0, D, step=16)
                    def _(c):
                        b0[r, pl.ds(c, 16)] = jnp.zeros((16,), jnp.float32)

                zbase = sid * ROWS_PAD

                @pl.loop(0, ROWS_PAD // CHUNK)
                def _(t):
                    pltpu.sync_copy(b0, agg.at[pl.ds(zbase + t * CHUNK,
                                                     CHUNK)])

            pltpu.make_async_copy(e_hbm.at[0, wid, pl.ds(p * PSTEPS, PSTEPS)],
                                  sidx, isem).wait()
            pltpu.make_async_copy(e_hbm.at[1, wid, pl.ds(p * PSTEPS, PSTEPS)],
                                  didx, isem).wait()
            if p == 0:
                plsc.subcore_barrier()

            for b in range(NBUF - 1):
                pltpu.async_copy(x_hbm.at[sidx.at[b]], bufs[b], gsem[b])

            @pl.loop(0, PSTEPS, step=NBUF)
            def _(t):
                for b in range(NBUF):
                    c = t + b
                    nb = (b + NBUF - 1) % NBUF
                    pltpu.make_async_copy(x_hbm.at[sidx.at[c]], bufs[b],
                                          gsem[b]).wait()
                    pltpu.async_copy(bufs[b], agg.at[didx.at[c]], ssem[b],
                                     add=True)

                    @pl.when(c + NBUF - 1 < PSTEPS)
                    def _():
                        @pl.when(c - 1 >= 0)
                        def _():
                            pltpu.make_async_copy(
                                bufs[nb], agg.at[didx.at[c - 1]],
                                ssem[nb]).wait()

                        pltpu.async_copy(x_hbm.at[sidx.at[c + NBUF - 1]],
                                         bufs[nb], gsem[nb])

            for kk in range(NBUF):
                c = PSTEPS - NBUF + kk
                pltpu.make_async_copy(bufs[c % NBUF], agg.at[didx.at[c]],
                                      ssem[c % NBUF]).wait()

        plsc.subcore_barrier()

        obase = sid * ROWS_PAD

        @pl.when(sid < NS - 1)
        def _():
            pltpu.sync_copy(agg.at[pl.ds(obase, ROWS_PAD)],
                            out_hbm.at[cid, pl.ds(obase, ROWS_PAD)])

        @pl.when(sid == NS - 1)
        def _():
            pltpu.sync_copy(agg.at[pl.ds(obase, N - (NS - 1) * ROWS_PAD)],
                            out_hbm.at[cid, pl.ds(obase, N - (NS - 1) * ROWS_PAD)])

    return k(x, e_all)


def _dot(a, b):
    return lax.dot_general(a, b, (((1,), (0,)), ((), ())),
                           preferred_element_type=jnp.float32,
                           precision=lax.Precision.DEFAULT)


def _phases01(p, i, eps_ref, o_ref, a0_ref, a1_ref, Wa_ref, ba_ref, gm_ref,
              bm_ref, Wb_ref, bb_ref, h1_ref, h2_ref, st_ref):
    @pl.when(jnp.logical_and(p == 0, i == 0))
    def _():
        st_ref[...] = jnp.zeros_like(st_ref)

    @pl.when(p == 0)
    def _():
        pre = (1.0 + eps_ref[0, 0]) * o_ref[...] + a0_ref[0] + a1_ref[0]
        h = _dot(pre, Wa_ref[...]) + ba_ref[...]
        h1_ref[pl.ds(i * BLK, BLK), :] = h
        st_ref[0:1, :] += jnp.sum(h, axis=0, keepdims=True)
        st_ref[1:2, :] += jnp.sum(h * h, axis=0, keepdims=True)

    @pl.when(p == 1)
    def _():
        mu = st_ref[0:1, :] * (1.0 / N)
        var = st_ref[1:2, :] * (1.0 / N) - mu * mu
        inv = lax.rsqrt(var + 1e-5)
        hn = (h1_ref[pl.ds(i * BLK, BLK), :] - mu) * (inv * gm_ref[...])
        hn = jnp.maximum(hn + bm_ref[...], 0.0)
        h2 = _dot(hn, Wb_ref[...]) + bb_ref[...]
        h2_ref[pl.ds(i * BLK, BLK), :] = h2
        st_ref[2:3, :] += jnp.sum(h2, axis=0, keepdims=True)
        st_ref[3:4, :] += jnp.sum(h2 * h2, axis=0, keepdims=True)


def _bn2(h2, st_ref, g_ref, b_ref):
    mu = st_ref[2:3, :] * (1.0 / N)
    var = st_ref[3:4, :] * (1.0 / N) - mu * mu
    inv = lax.rsqrt(var + 1e-5)
    return jnp.maximum((h2 - mu) * (inv * g_ref[...]) + b_ref[...], 0.0)


def _layer0_body(eps_ref, o_ref, a0_ref, a1_ref, Wa_ref, ba_ref, gm_ref,
                 bm_ref, Wb_ref, bb_ref, g_ref, b_ref, out_ref,
                 h1_ref, h2_ref, st_ref):
    p, i = pl.program_id(0), pl.program_id(1)
    _phases01(p, i, eps_ref, o_ref, a0_ref, a1_ref, Wa_ref, ba_ref, gm_ref,
              bm_ref, Wb_ref, bb_ref, h1_ref, h2_ref, st_ref)

    @pl.when(p == 2)
    def _():
        out_ref[...] = _bn2(h2_ref[pl.ds(i * BLK, BLK), :], st_ref,
                            g_ref, b_ref)


def _layer1_body(eps_ref, base_ref, o_ref, a0_ref, a1_ref, Wa_ref, ba_ref,
                 gm_ref, bm_ref, Wb_ref, bb_ref, g_ref, b_ref, out_ref,
                 h1_ref, h2_ref, st_ref):
    del base_ref
    p, i = pl.program_id(0), pl.program_id(1)
    _phases01(p, i, eps_ref, o_ref, a0_ref, a1_ref, Wa_ref, ba_ref, gm_ref,
              bm_ref, Wb_ref, bb_ref, h1_ref, h2_ref, st_ref)

    @pl.when(p == 2)
    def _():
        out_ref[...] = _bn2(h2_ref[pl.ds(i * BLK, BLK), :], st_ref,
                            g_ref, b_ref)


def _concat01_body(x_ref, o_ref, out_ref):
    j = pl.program_id(0)

    @pl.when(j == 0)
    def _():
        out_ref[...] = x_ref[...]

    @pl.when(j == 1)
    def _():
        out_ref[...] = o_ref[...]


def _concat01(x, o1):
    return pl.pallas_call(
        _concat01_body,
        grid=(2, NBLK),
        in_specs=[
            pl.BlockSpec((BLK, D), lambda j, i: (jnp.where(j == 0, i, 0), 0)),
            pl.BlockSpec((BLK, D), lambda j, i: (jnp.where(j == 1, i, 0), 0)),
        ],
        out_specs=pl.BlockSpec((BLK, D), lambda j, i: (i, j)),
        out_shape=jax.ShapeDtypeStruct((N, 3 * D), jnp.float32),
    )(x, o1)


def _vspec(shape):
    return pl.BlockSpec(shape, lambda p, i: tuple(0 for _ in shape))


_scratch = [pltpu.VMEM((N, D), jnp.float32),
            pltpu.VMEM((N, D), jnp.float32),
            pltpu.VMEM((8, D), jnp.float32)]


def _prep(eps, vecs):
    return (jnp.reshape(eps, (1, 1)),) + tuple(v.reshape(1, D) for v in vecs)


def _agg_spec(c):
    return pl.BlockSpec((1, BLK, D),
                        lambda p, i: (c, jnp.where(p == 0, i, 0), 0))


def _gin_layer0(o, aggp, eps, Wa, ba, gm, bm, Wb, bb, g, b):
    eps2, ba2, gm2, bm2, bb2, g2, b2 = _prep(eps, (ba, gm, bm, bb, g, b))
    p0_spec = pl.BlockSpec((BLK, D), lambda p, i: (jnp.where(p == 0, i, 0), 0))
    return pl.pallas_call(
        _layer0_body,
        grid=(3, NBLK),
        in_specs=[pl.BlockSpec(memory_space=pltpu.SMEM),
                  p0_spec, _agg_spec(0), _agg_spec(1),
                  _vspec((D, D)), _vspec((1, D)), _vspec((1, D)),
                  _vspec((1, D)), _vspec((D, D)), _vspec((1, D)),
                  _vspec((1, D)), _vspec((1, D))],
        out_specs=pl.BlockSpec((BLK, D),
                               lambda p, i: (jnp.where(p == 2, i, 0), 0)),
        out_shape=jax.ShapeDtypeStruct((N, D), jnp.float32),
        scratch_shapes=_scratch,
    )(eps2, o, aggp, aggp, Wa, ba2, gm2, bm2, Wb, bb2, g2, b2)


def _gin_layer1(base, o, aggp, eps, Wa, ba, gm, bm, Wb, bb, g, b):
    eps2, ba2, gm2, bm2, bb2, g2, b2 = _prep(eps, (ba, gm, bm, bb, g, b))
    p0_spec = pl.BlockSpec((BLK, D), lambda p, i: (jnp.where(p == 0, i, 0), 0))
    return pl.pallas_call(
        _layer1_body,
        grid=(3, NBLK),
        in_specs=[pl.BlockSpec(memory_space=pltpu.SMEM),
                  pl.BlockSpec((8, D), lambda p, i: (0, 0)),
                  p0_spec, _agg_spec(0), _agg_spec(1),
                  _vspec((D, D)), _vspec((1, D)), _vspec((1, D)),
                  _vspec((1, D)), _vspec((D, D)), _vspec((1, D)),
                  _vspec((1, D)), _vspec((1, D))],
        out_specs=pl.BlockSpec((BLK, D),
                               lambda p, i: (jnp.where(p == 2, i, 0), 2)),
        out_shape=jax.ShapeDtypeStruct((N, 3 * D), jnp.float32),
        input_output_aliases={1: 0},
        scratch_shapes=_scratch,
    )(eps2, base, o, aggp, aggp, Wa, ba2, gm2, bm2, Wb, bb2, g2, b2)


def kernel(x, edge_index, eps0, Wa0, ba0, gm0, bm0, Wb0, bb0, g0, b0,
           eps1, Wa1, ba1, gm1, bm1, Wb1, bb1, g1, b1):
    e_all = jnp.concatenate(
        [edge_index.astype(jnp.int32), jnp.asarray(_PAD_EDGES)],
        axis=1).reshape(2, NW, STEPS, CHUNK)

    aggp0 = _sc_agg(x, e_all)
    o1 = _gin_layer0(x, aggp0, eps0, Wa0, ba0, gm0, bm0, Wb0, bb0, g0, b0)
    aggp1 = _sc_agg(o1, e_all)
    base = _concat01(x, o1)
    return _gin_layer1(base, o1, aggp1, eps1, Wa1, ba1, gm1, bm1, Wb1, bb1,
                       g1, b1)
